# baseline (speedup 1.0000x reference)
import functools
import math

import jax
import jax.numpy as jnp
from jax import lax
from jax.experimental import pallas as pl
from jax.experimental.pallas import tpu as pltpu
from jax.experimental.pallas import tpu_sc as plsc

B = 4
N = 10000
E = 160000
D = 128
H = 128
AD = 32

NC = 2
NS = 16
CH = 64
KC = 160
NST = 4
HC = KC // NST
EPB = NS * KC * CH
ACC_ROWS = 10112
S_ROWS = 10112
OR = S_ROWS // NS
OCH = (CH,) * (OR // CH) + (OR % CH,)

NB = 2000


def _sc_aggregate_call(nodes_flat, sidx, didx):
    mesh = plsc.VectorSubcoreMesh(core_axis_name="c", subcore_axis_name="s")

    @functools.partial(
        pl.kernel,
        out_type=jax.ShapeDtypeStruct((B, S_ROWS, D), jnp.float32),
        mesh=mesh,
        scratch_types=[
            pltpu.VMEM((HC + 1, CH), jnp.int32),
            pltpu.VMEM((HC + 1, CH), jnp.int32),
            pltpu.VMEM((CH, D), jnp.float32),
            pltpu.VMEM((CH, D), jnp.float32),
            pltpu.VMEM((CH, D), jnp.float32),
            pltpu.VMEM((CH, D), jnp.float32),
            pltpu.VMEM_SHARED((ACC_ROWS, D), jnp.float32),
            pltpu.SemaphoreType.DMA,
            pltpu.SemaphoreType.DMA,
            pltpu.SemaphoreType.DMA,
            pltpu.SemaphoreType.DMA,
        ],
    )
    def agg(nodes, sidx_h, didx_h, out, sidx_v, didx_v, rb0, rb1, rb2, rb3,
            acc, sem0, sem1, sem2, sem3):
        c = lax.axis_index("c")
        s = lax.axis_index("s")
        rbs = (rb0, rb1, rb2, rb3)
        sems = (sem0, sem1, sem2, sem3)

        def zrow(i, carry):
            for j in range(D // 16):
                rb3[i, pl.ds(j * 16, 16)] = jnp.zeros((16,), jnp.float32)
            return carry

        def step(j, jn, nb):
            kb = (nb + 3) % 4
            pltpu.async_copy(nodes.at[sidx_v.at[jn]], rbs[kb], sems[kb])
            pltpu.make_async_copy(nodes.at[sidx_v.at[j]], rbs[nb],
                                  sems[nb]).wait()
            pltpu.sync_copy(rbs[nb], acc.at[didx_v.at[j]], add=True)

        for bi in range(B // NC):
            b = NC * c + bi
            lax.fori_loop(0, CH, zrow, 0)
            for z, sz in enumerate(OCH):
                pltpu.sync_copy(rb3.at[pl.ds(0, sz)],
                                acc.at[pl.ds(s * OR + z * CH, sz)])
            plsc.subcore_barrier()

            for hf in range(NST):
                pltpu.sync_copy(sidx_h.at[b, s, hf], sidx_v)
                pltpu.sync_copy(didx_h.at[b, s, hf], didx_v)

                for k in range(3):
                    pltpu.async_copy(nodes.at[sidx_v.at[k]], rbs[k], sems[k])

                def quad(t, carry):
                    for nb in range(4):
                        j = 4 * t + nb
                        step(j, j + 3, nb)
                    return carry

                lax.fori_loop(0, HC // 4 - 1, quad, 0)
                for j in range(HC - 4, HC):
                    step(j, min(j + 3, HC), j % 4)
                for k in range(3):
                    kb = (HC + k) % 4
                    pltpu.make_async_copy(nodes.at[sidx_v.at[HC]], rbs[kb],
                                          sems[kb]).wait()
            plsc.subcore_barrier()

            for z, sz in enumerate(OCH):
                r0 = s * OR + z * CH
                pltpu.sync_copy(acc.at[pl.ds(r0, sz)], rb0.at[pl.ds(0, sz)])
                pltpu.sync_copy(rb0.at[pl.ds(0, sz)], out.at[b, pl.ds(r0, sz)])
            plsc.subcore_barrier()

    return agg(nodes_flat, sidx, didx)


def _tc_body(s_ref, x_ref, wm_ref, ws_ref, b_ref, g_ref, be_ref, ad_ref,
             wq_ref, bq_ref, wk_ref, bk_ref, o_ref):
    f32 = jnp.float32
    sblk = s_ref[0]
    xblk = x_ref[0]
    pre = (jnp.dot(sblk, wm_ref[...], preferred_element_type=f32)
           + jnp.dot(xblk, ws_ref[...], preferred_element_type=f32)
           + b_ref[...])
    h = jnp.maximum(pre, 0.0)
    mu = jnp.mean(h, axis=1, keepdims=True)
    d = h - mu
    var = jnp.mean(d * d, axis=1, keepdims=True)

    q = jnp.dot(ad_ref[0], wq_ref[...], preferred_element_type=f32) + bq_ref[...]
    u = lax.dot_general(q, wk_ref[...], (((1,), (1,)), ((), ())),
                        preferred_element_type=f32)
    g = u * g_ref[...]
    sg = jnp.sum(g)
    bu = jnp.sum(be_ref[...] * u)
    cq = jnp.sum(bk_ref[...] * q)

    t = lax.dot_general(h, g, (((1,), (1,)), ((), ())),
                        preferred_element_type=f32)
    inv = lax.rsqrt(var + 1e-5)
    o_ref[0] = ((t - mu * sg) * inv + (bu + cq)) * (1.0 / math.sqrt(float(H)))


def _tc_dense_call(S, X, W_msg, W_self, b, gamma, beta, ad, Wq, bq, Wk, bk):
    row = lambda i, j: (0, 0)
    return pl.pallas_call(
        _tc_body,
        grid=(B, N // NB),
        in_specs=[
            pl.BlockSpec((1, NB, D), lambda i, j: (i, j, 0)),
            pl.BlockSpec((1, NB, D), lambda i, j: (i, j, 0)),
            pl.BlockSpec((D, H), row),
            pl.BlockSpec((D, H), row),
            pl.BlockSpec((1, H), row),
            pl.BlockSpec((1, H), row),
            pl.BlockSpec((1, H), row),
            pl.BlockSpec((1, 1, AD), lambda i, j: (i, 0, 0)),
            pl.BlockSpec((AD, H), row),
            pl.BlockSpec((1, H), row),
            pl.BlockSpec((H, H), row),
            pl.BlockSpec((1, H), row),
        ],
        out_specs=pl.BlockSpec((1, NB, 1), lambda i, j: (i, j, 0)),
        out_shape=jax.ShapeDtypeStruct((B, N, 1), jnp.float32),
    )(S, X, W_msg, W_self, b, gamma, beta, ad, Wq, bq, Wk, bk)


def kernel(graph_nodes, graph_edge_links, mask, current_ad,
           W_msg, W_self, b, gamma, beta, Wq, bq, Wk, bk):
    src = graph_edge_links[:, 0, :]
    dst = graph_edge_links[:, 1, :]
    npad = EPB - E
    boff = (jnp.arange(B, dtype=jnp.int32) * N)[:, None]
    pad_s = (jnp.arange(npad, dtype=jnp.int32) * 97) % N
    pad_d = N + (jnp.arange(npad, dtype=jnp.int32) % (ACC_ROWS - N))
    fsrc = jnp.concatenate([src, jnp.broadcast_to(pad_s, (B, npad))], 1) + boff
    fdst = jnp.concatenate([dst, jnp.broadcast_to(pad_d, (B, npad))], 1)
    dum_s = (jnp.arange(CH, dtype=jnp.int32) * 131) % N
    dum_s = jnp.broadcast_to(dum_s, (B, NS, NST, 1, CH)) + boff[:, None, None,
                                                                None]
    dum_d = N + jnp.broadcast_to(jnp.arange(CH, dtype=jnp.int32) %
                                 (ACC_ROWS - N), (B, NS, NST, 1, CH))
    sidx = jnp.concatenate([fsrc.reshape(B, NS, NST, HC, CH), dum_s], axis=3)
    didx = jnp.concatenate([fdst.reshape(B, NS, NST, HC, CH), dum_d], axis=3)
    nodes_flat = graph_nodes.reshape(B * N, D)

    S = _sc_aggregate_call(nodes_flat, sidx, didx)

    logits = _tc_dense_call(
        S, graph_nodes, W_msg, W_self,
        b.reshape(1, H), gamma.reshape(1, H), beta.reshape(1, H),
        current_ad.reshape(B, 1, AD), Wq, bq.reshape(1, H), Wk,
        bk.reshape(1, H),
    ).reshape(B, N)
    return jnp.where(mask, logits, jnp.float32(-1e9))


# speedup vs baseline: 1.1095x; 1.1095x over previous
import functools
import math

import jax
import jax.numpy as jnp
from jax import lax
from jax.experimental import pallas as pl
from jax.experimental.pallas import tpu as pltpu
from jax.experimental.pallas import tpu_sc as plsc

B = 4
N = 10000
E = 160000
D = 128
H = 128
AD = 32

NC = 2
NS = 16
CH = 64
KC = 160
HC = KC // 2
EPB = NS * KC * CH
ACC_ROWS = 10112
S_ROWS = 10112
OR = S_ROWS // NS
OCH = (CH,) * (OR // CH) + (OR % CH,)

NB = 5000


def _sc_aggregate_call(nodes_flat, sidx, didx):
    mesh = plsc.VectorSubcoreMesh(core_axis_name="c", subcore_axis_name="s")

    @functools.partial(
        pl.kernel,
        out_type=jax.ShapeDtypeStruct((B, S_ROWS, D), jnp.float32),
        mesh=mesh,
        scratch_types=[
            pltpu.VMEM((HC + 1, CH), jnp.int32),
            pltpu.VMEM((HC + 1, CH), jnp.int32),
            pltpu.VMEM((CH, D), jnp.float32),
            pltpu.VMEM((CH, D), jnp.float32),
            pltpu.VMEM((CH, D), jnp.float32),
            pltpu.VMEM_SHARED((ACC_ROWS, D), jnp.float32),
            pltpu.SemaphoreType.DMA,
            pltpu.SemaphoreType.DMA,
            pltpu.SemaphoreType.DMA,
        ],
    )
    def agg(nodes, sidx_h, didx_h, out, sidx_v, didx_v, rb0, rb1, rb2,
            acc, sem0, sem1, sem2):
        c = lax.axis_index("c")
        s = lax.axis_index("s")
        rbs = (rb0, rb1, rb2)
        sems = (sem0, sem1, sem2)

        def zrow(i, carry):
            for j in range(D // 16):
                rb2[i, pl.ds(j * 16, 16)] = jnp.zeros((16,), jnp.float32)
            return carry

        def step(j, jn, nb):
            kb = (nb + 2) % 3
            pltpu.async_copy(nodes.at[sidx_v.at[jn]], rbs[kb], sems[kb])
            pltpu.make_async_copy(nodes.at[sidx_v.at[j]], rbs[nb],
                                  sems[nb]).wait()
            pltpu.sync_copy(rbs[nb], acc.at[didx_v.at[j]], add=True)

        for bi in range(B // NC):
            b = NC * c + bi
            lax.fori_loop(0, CH, zrow, 0)
            for z, sz in enumerate(OCH):
                pltpu.sync_copy(rb2.at[pl.ds(0, sz)],
                                acc.at[pl.ds(s * OR + z * CH, sz)])
            plsc.subcore_barrier()

            for hf in range(2):
                pltpu.sync_copy(sidx_h.at[b, s, hf], sidx_v)
                pltpu.sync_copy(didx_h.at[b, s, hf], didx_v)

                for k in range(2):
                    pltpu.async_copy(nodes.at[sidx_v.at[k]], rbs[k], sems[k])

                def triple(t, carry):
                    for nb in range(3):
                        j = 3 * t + nb
                        step(j, j + 2, nb)
                    return carry

                lax.fori_loop(0, HC // 3, triple, 0)
                step(HC - 2, HC, (HC - 2) % 3)
                step(HC - 1, HC, (HC - 1) % 3)
                for k in range(2):
                    kb = (HC + k) % 3
                    pltpu.make_async_copy(nodes.at[sidx_v.at[HC]], rbs[kb],
                                          sems[kb]).wait()
            plsc.subcore_barrier()

            for z, sz in enumerate(OCH):
                r0 = s * OR + z * CH
                pltpu.sync_copy(acc.at[pl.ds(r0, sz)], rb0.at[pl.ds(0, sz)])
                pltpu.sync_copy(rb0.at[pl.ds(0, sz)], out.at[b, pl.ds(r0, sz)])
            plsc.subcore_barrier()

    return agg(nodes_flat, sidx, didx)


def _tc_body(s_ref, x_ref, wm_ref, ws_ref, b_ref, g_ref, be_ref, ad_ref,
             wq_ref, bq_ref, wk_ref, bk_ref, o_ref):
    f32 = jnp.float32
    sblk = s_ref[0]
    xblk = x_ref[0]
    pre = (jnp.dot(sblk, wm_ref[...], preferred_element_type=f32)
           + jnp.dot(xblk, ws_ref[...], preferred_element_type=f32)
           + b_ref[...])
    h = jnp.maximum(pre, 0.0)
    mu = jnp.mean(h, axis=1, keepdims=True)
    d = h - mu
    var = jnp.mean(d * d, axis=1, keepdims=True)

    q = jnp.dot(ad_ref[0], wq_ref[...], preferred_element_type=f32) + bq_ref[...]
    u = lax.dot_general(q, wk_ref[...], (((1,), (1,)), ((), ())),
                        preferred_element_type=f32)
    g = u * g_ref[...]
    sg = jnp.sum(g)
    bu = jnp.sum(be_ref[...] * u)
    cq = jnp.sum(bk_ref[...] * q)

    t = lax.dot_general(h, g, (((1,), (1,)), ((), ())),
                        preferred_element_type=f32)
    inv = lax.rsqrt(var + 1e-5)
    o_ref[0] = ((t - mu * sg) * inv + (bu + cq)) * (1.0 / math.sqrt(float(H)))


def _tc_dense_call(S, X, W_msg, W_self, b, gamma, beta, ad, Wq, bq, Wk, bk):
    row = lambda i, j: (0, 0)
    return pl.pallas_call(
        _tc_body,
        grid=(B, N // NB),
        in_specs=[
            pl.BlockSpec((1, NB, D), lambda i, j: (i, j, 0)),
            pl.BlockSpec((1, NB, D), lambda i, j: (i, j, 0)),
            pl.BlockSpec((D, H), row),
            pl.BlockSpec((D, H), row),
            pl.BlockSpec((1, H), row),
            pl.BlockSpec((1, H), row),
            pl.BlockSpec((1, H), row),
            pl.BlockSpec((1, 1, AD), lambda i, j: (i, 0, 0)),
            pl.BlockSpec((AD, H), row),
            pl.BlockSpec((1, H), row),
            pl.BlockSpec((H, H), row),
            pl.BlockSpec((1, H), row),
        ],
        out_specs=pl.BlockSpec((1, NB, 1), lambda i, j: (i, j, 0)),
        out_shape=jax.ShapeDtypeStruct((B, N, 1), jnp.float32),
    )(S, X, W_msg, W_self, b, gamma, beta, ad, Wq, bq, Wk, bk)


def kernel(graph_nodes, graph_edge_links, mask, current_ad,
           W_msg, W_self, b, gamma, beta, Wq, bq, Wk, bk):
    src = graph_edge_links[:, 0, :]
    dst = graph_edge_links[:, 1, :]
    npad = EPB - E
    boff = (jnp.arange(B, dtype=jnp.int32) * N)[:, None]
    pad_s = (jnp.arange(npad, dtype=jnp.int32) * 97) % N
    pad_d = N + (jnp.arange(npad, dtype=jnp.int32) % (ACC_ROWS - N))
    fsrc = jnp.concatenate([src, jnp.broadcast_to(pad_s, (B, npad))], 1) + boff
    fdst = jnp.concatenate([dst, jnp.broadcast_to(pad_d, (B, npad))], 1)
    dum_s = (jnp.arange(CH, dtype=jnp.int32) * 131) % N
    dum_s = jnp.broadcast_to(dum_s, (B, NS, 2, 1, CH)) + boff[:, None, None,
                                                             None]
    dum_d = N + jnp.broadcast_to(jnp.arange(CH, dtype=jnp.int32) %
                                 (ACC_ROWS - N), (B, NS, 2, 1, CH))
    sidx = jnp.concatenate([fsrc.reshape(B, NS, 2, HC, CH), dum_s], axis=3)
    didx = jnp.concatenate([fdst.reshape(B, NS, 2, HC, CH), dum_d], axis=3)
    nodes_flat = graph_nodes.reshape(B * N, D)

    S = _sc_aggregate_call(nodes_flat, sidx, didx)

    logits = _tc_dense_call(
        S, graph_nodes, W_msg, W_self,
        b.reshape(1, H), gamma.reshape(1, H), beta.reshape(1, H),
        current_ad.reshape(B, 1, AD), Wq, bq.reshape(1, H), Wk,
        bk.reshape(1, H),
    ).reshape(B, N)
    return jnp.where(mask, logits, jnp.float32(-1e9))


# speedup vs baseline: 1.1460x; 1.0329x over previous
import functools
import math

import jax
import jax.numpy as jnp
from jax import lax
from jax.experimental import pallas as pl
from jax.experimental.pallas import tpu as pltpu
from jax.experimental.pallas import tpu_sc as plsc

B = 4
N = 10000
E = 160000
D = 128
H = 128
AD = 32

NC = 2
NS = 16
CH = 80
KC = 128
HC = KC // 2
EPB = NS * KC * CH
ACC_ROWS = 10112
S_ROWS = 10112
OR = S_ROWS // NS
OCH = (CH,) * (OR // CH) + (OR % CH,)

NB = 5000


def _sc_aggregate_call(nodes_flat, sidx, didx):
    mesh = plsc.VectorSubcoreMesh(core_axis_name="c", subcore_axis_name="s")

    @functools.partial(
        pl.kernel,
        out_type=jax.ShapeDtypeStruct((B, S_ROWS, D), jnp.float32),
        mesh=mesh,
        scratch_types=[
            pltpu.VMEM((HC + 1, CH), jnp.int32),
            pltpu.VMEM((HC + 1, CH), jnp.int32),
            pltpu.VMEM((CH, D), jnp.float32),
            pltpu.VMEM((CH, D), jnp.float32),
            pltpu.VMEM((CH, D), jnp.float32),
            pltpu.VMEM_SHARED((ACC_ROWS, D), jnp.float32),
            pltpu.SemaphoreType.DMA,
            pltpu.SemaphoreType.DMA,
            pltpu.SemaphoreType.DMA,
        ],
    )
    def agg(nodes, sidx_h, didx_h, out, sidx_v, didx_v, rb0, rb1, rb2,
            acc, sem0, sem1, sem2):
        c = lax.axis_index("c")
        s = lax.axis_index("s")
        rbs = (rb0, rb1, rb2)
        sems = (sem0, sem1, sem2)

        def zrow(i, carry):
            for j in range(D // 16):
                rb2[i, pl.ds(j * 16, 16)] = jnp.zeros((16,), jnp.float32)
            return carry

        def step(j, jn, nb):
            kb = (nb + 2) % 3
            pltpu.async_copy(nodes.at[sidx_v.at[jn]], rbs[kb], sems[kb])
            pltpu.make_async_copy(nodes.at[sidx_v.at[j]], rbs[nb],
                                  sems[nb]).wait()
            pltpu.sync_copy(rbs[nb], acc.at[didx_v.at[j]], add=True)

        for bi in range(B // NC):
            b = NC * c + bi
            lax.fori_loop(0, CH, zrow, 0)
            for z, sz in enumerate(OCH):
                pltpu.sync_copy(rb2.at[pl.ds(0, sz)],
                                acc.at[pl.ds(s * OR + z * CH, sz)])
            plsc.subcore_barrier()

            for hf in range(2):
                pltpu.sync_copy(sidx_h.at[b, s, hf], sidx_v)
                pltpu.sync_copy(didx_h.at[b, s, hf], didx_v)

                for k in range(2):
                    pltpu.async_copy(nodes.at[sidx_v.at[k]], rbs[k], sems[k])

                def triple(t, carry):
                    for nb in range(3):
                        j = 3 * t + nb
                        step(j, j + 2, nb)
                    return carry

                nt = (HC - 2) // 3
                lax.fori_loop(0, nt, triple, 0)
                for j in range(3 * nt, HC):
                    step(j, min(j + 2, HC), j % 3)
                for k in range(2):
                    kb = (HC + k) % 3
                    pltpu.make_async_copy(nodes.at[sidx_v.at[HC]], rbs[kb],
                                          sems[kb]).wait()
            plsc.subcore_barrier()

            for z, sz in enumerate(OCH):
                r0 = s * OR + z * CH
                pltpu.sync_copy(acc.at[pl.ds(r0, sz)], rb0.at[pl.ds(0, sz)])
                pltpu.sync_copy(rb0.at[pl.ds(0, sz)], out.at[b, pl.ds(r0, sz)])
            plsc.subcore_barrier()

    return agg(nodes_flat, sidx, didx)


def _tc_body(s_ref, x_ref, wm_ref, ws_ref, b_ref, g_ref, be_ref, ad_ref,
             wq_ref, bq_ref, wk_ref, bk_ref, o_ref):
    f32 = jnp.float32
    sblk = s_ref[0]
    xblk = x_ref[0]
    pre = (jnp.dot(sblk, wm_ref[...], preferred_element_type=f32)
           + jnp.dot(xblk, ws_ref[...], preferred_element_type=f32)
           + b_ref[...])
    h = jnp.maximum(pre, 0.0)
    mu = jnp.mean(h, axis=1, keepdims=True)
    d = h - mu
    var = jnp.mean(d * d, axis=1, keepdims=True)

    q = jnp.dot(ad_ref[0], wq_ref[...], preferred_element_type=f32) + bq_ref[...]
    u = lax.dot_general(q, wk_ref[...], (((1,), (1,)), ((), ())),
                        preferred_element_type=f32)
    g = u * g_ref[...]
    sg = jnp.sum(g)
    bu = jnp.sum(be_ref[...] * u)
    cq = jnp.sum(bk_ref[...] * q)

    t = lax.dot_general(h, g, (((1,), (1,)), ((), ())),
                        preferred_element_type=f32)
    inv = lax.rsqrt(var + 1e-5)
    o_ref[0] = ((t - mu * sg) * inv + (bu + cq)) * (1.0 / math.sqrt(float(H)))


def _tc_dense_call(S, X, W_msg, W_self, b, gamma, beta, ad, Wq, bq, Wk, bk):
    row = lambda i, j: (0, 0)
    return pl.pallas_call(
        _tc_body,
        grid=(B, N // NB),
        in_specs=[
            pl.BlockSpec((1, NB, D), lambda i, j: (i, j, 0)),
            pl.BlockSpec((1, NB, D), lambda i, j: (i, j, 0)),
            pl.BlockSpec((D, H), row),
            pl.BlockSpec((D, H), row),
            pl.BlockSpec((1, H), row),
            pl.BlockSpec((1, H), row),
            pl.BlockSpec((1, H), row),
            pl.BlockSpec((1, 1, AD), lambda i, j: (i, 0, 0)),
            pl.BlockSpec((AD, H), row),
            pl.BlockSpec((1, H), row),
            pl.BlockSpec((H, H), row),
            pl.BlockSpec((1, H), row),
        ],
        out_specs=pl.BlockSpec((1, NB, 1), lambda i, j: (i, j, 0)),
        out_shape=jax.ShapeDtypeStruct((B, N, 1), jnp.float32),
    )(S, X, W_msg, W_self, b, gamma, beta, ad, Wq, bq, Wk, bk)


def kernel(graph_nodes, graph_edge_links, mask, current_ad,
           W_msg, W_self, b, gamma, beta, Wq, bq, Wk, bk):
    src = graph_edge_links[:, 0, :]
    dst = graph_edge_links[:, 1, :]
    npad = EPB - E
    boff = (jnp.arange(B, dtype=jnp.int32) * N)[:, None]
    pad_s = (jnp.arange(npad, dtype=jnp.int32) * 97) % N
    pad_d = N + (jnp.arange(npad, dtype=jnp.int32) % (ACC_ROWS - N))
    fsrc = jnp.concatenate([src, jnp.broadcast_to(pad_s, (B, npad))], 1) + boff
    fdst = jnp.concatenate([dst, jnp.broadcast_to(pad_d, (B, npad))], 1)
    dum_s = (jnp.arange(CH, dtype=jnp.int32) * 131) % N
    dum_s = jnp.broadcast_to(dum_s, (B, NS, 2, 1, CH)) + boff[:, None, None,
                                                             None]
    dum_d = N + jnp.broadcast_to(jnp.arange(CH, dtype=jnp.int32) %
                                 (ACC_ROWS - N), (B, NS, 2, 1, CH))
    sidx = jnp.concatenate([fsrc.reshape(B, NS, 2, HC, CH), dum_s], axis=3)
    didx = jnp.concatenate([fdst.reshape(B, NS, 2, HC, CH), dum_d], axis=3)
    nodes_flat = graph_nodes.reshape(B * N, D)

    S = _sc_aggregate_call(nodes_flat, sidx, didx)

    logits = _tc_dense_call(
        S, graph_nodes, W_msg, W_self,
        b.reshape(1, H), gamma.reshape(1, H), beta.reshape(1, H),
        current_ad.reshape(B, 1, AD), Wq, bq.reshape(1, H), Wk,
        bk.reshape(1, H),
    ).reshape(B, N)
    return jnp.where(mask, logits, jnp.float32(-1e9))
